# baseline (speedup 1.0000x reference)
import jax
import jax.numpy as jnp
from jax.experimental import pallas as pl

_B, _C, _F, _H, _W = 8, 3, 64, 128, 128
_BC = _B * _C
_HW = _H * _W
_KEEP = 32


def _pool_body(x_ref, cw_ref, out_ref, outc_ref):
    i = pl.program_id(0)
    x = x_ref[0]
    s = jnp.sum(x, axis=-1) * (1.0 / _HW)
    prod = jax.lax.dot(cw_ref[...].astype(jnp.bfloat16),
                       x.astype(jnp.bfloat16),
                       preferred_element_type=jnp.float32)
    sc = jnp.sum(prod, axis=-1) * (1.0 / _HW)
    out_ref[pl.ds(i, 1), :] = s.reshape(1, _F)
    outc_ref[pl.ds(i, 1), :] = sc.reshape(1, _F)


def _score_body(pooled_ref, xcv_ref, conv_b_ref, fc_wT_ref,
                fc_b_ref, M1_ref, b1_ref, GG_ref, Mr_ref, br_ref, Ml_ref,
                bl_ref, gamr_ref, betr_ref, A8_ref, B8_ref, P_ref, Q_ref,
                a_ref, mask_ref):
    hi = jax.lax.Precision.HIGHEST
    bf = jnp.bfloat16
    f32 = jnp.float32

    def dot(a, b):
        return jax.lax.dot(a, b, precision=hi)

    def dotb(a, b):
        return jax.lax.dot(a.astype(bf), b.astype(bf),
                           preferred_element_type=f32)

    p = pooled_ref[...]
    x_conv = xcv_ref[...] + conv_b_ref[...]
    scores = jax.nn.sigmoid(dotb(x_conv, fc_wT_ref[...]) + fc_b_ref[...])

    r8 = dot(p, A8_ref[...])
    c8 = dot(p, B8_ref[...])
    xr0 = dotb(M1_ref[...], r8) + b1_ref[...]
    xc0 = dotb(M1_ref[...], c8) + b1_ref[...]

    rs = jnp.sum(xr0, axis=1, keepdims=True) + jnp.sum(xc0, axis=1, keepdims=True)
    mur = dot(GG_ref[...], rs) * (1.0 / 128.0)
    dr = xr0 - mur
    dc = xc0 - mur
    rs2 = (jnp.sum(dr * dr, axis=1, keepdims=True)
           + jnp.sum(dc * dc, axis=1, keepdims=True))
    varr = dot(GG_ref[...], rs2) * (1.0 / 128.0)
    inv = gamr_ref[...] / jnp.sqrt(varr + 1e-5)
    sr = jax.nn.sigmoid(dr * inv + betr_ref[...])
    sc = jax.nn.sigmoid(dc * inv + betr_ref[...])

    ar = jax.nn.sigmoid(dotb(Mr_ref[...], sr) + br_ref[...])
    al = jax.nn.sigmoid(dotb(Ml_ref[...], sc) + bl_ref[...])
    att = dot(ar, P_ref[...]) * dot(al, Q_ref[...])

    a = a_ref[0, 0]
    fin = a * att + (1.0 - a) * scores

    ff = fin[:, :, None]
    fg = fin[:, None, :]
    io_f = jax.lax.broadcasted_iota(jnp.int32, (_BC, _F, _F), 1)
    io_g = jax.lax.broadcasted_iota(jnp.int32, (_BC, _F, _F), 2)
    beats = (fg > ff) | ((fg == ff) & (io_g < io_f))
    cnt = jnp.sum(beats.astype(jnp.float32), axis=2)
    mask_ref[...] = (cnt < float(_KEEP)).astype(jnp.float32)


def _mul_body(x_ref, m_ref, o1_ref, o2_ref):
    o1_ref[...] = x_ref[...] * m_ref[...]
    o2_ref[...] = jnp.zeros_like(o2_ref)


def kernel(x_freq, conv_w, conv_b, conv1_w, conv1_b, convr_w, convr_b,
           convl_w, convl_b, bn_gamma, bn_beta, fc_w, fc_b, a_param):
    f32 = jnp.float32
    xf = x_freq.reshape(_BC, _F, _HW)

    pooled, xconv = pl.pallas_call(
        _pool_body,
        grid=(_BC,),
        in_specs=[pl.BlockSpec((1, _F, _HW), lambda i: (i, 0, 0)),
                  pl.BlockSpec((_F, _F), lambda i: (0, 0))],
        out_specs=[pl.BlockSpec((_BC, _F), lambda i: (0, 0)),
                   pl.BlockSpec((_BC, _F), lambda i: (0, 0))],
        out_shape=[jax.ShapeDtypeStruct((_BC, _F), f32),
                   jax.ShapeDtypeStruct((_BC, _F), f32)],
    )(xf, conv_w)

    eyeB = jnp.eye(_B, dtype=f32)
    M1 = jnp.kron(eyeB, conv1_w)
    Mr = jnp.kron(eyeB, convr_w)
    Ml = jnp.kron(eyeB, convl_w)
    b1 = jnp.tile(conv1_b, _B).reshape(_BC, 1)
    br = jnp.tile(convr_b, _B).reshape(_BC, 1)
    bl = jnp.tile(convl_b, _B).reshape(_BC, 1)
    gamr = jnp.tile(bn_gamma, _B).reshape(_BC, 1)
    betr = jnp.tile(bn_beta, _B).reshape(_BC, 1)
    ch = jnp.arange(_BC) % _C
    GG = (ch[:, None] == ch[None, :]).astype(f32)
    q8 = jnp.arange(_F, dtype=jnp.int32)
    A8 = ((q8[:, None] // 8) == jnp.arange(8)[None, :]).astype(f32) / 8.0
    B8 = ((q8[:, None] % 8) == jnp.arange(8)[None, :]).astype(f32) / 8.0
    P = (jnp.arange(8)[:, None] == (q8[None, :] // 8)).astype(f32)
    Q = (jnp.arange(8)[:, None] == (q8[None, :] % 8)).astype(f32)

    mask = pl.pallas_call(
        _score_body,
        out_shape=jax.ShapeDtypeStruct((_BC, _F), f32),
    )(pooled, xconv, conv_b.reshape(1, _F), fc_w.T, fc_b.reshape(1, _F),
      M1, b1, GG, Mr, br, Ml, bl, gamr, betr, A8, B8, P, Q,
      jnp.asarray(a_param, f32).reshape(1, 1))

    mask3 = mask.reshape(_BC, _F, 1)
    out1, out2 = pl.pallas_call(
        _mul_body,
        grid=(_BC,),
        in_specs=[pl.BlockSpec((1, _F, _HW), lambda i: (i, 0, 0)),
                  pl.BlockSpec((1, _F, 1), lambda i: (i, 0, 0))],
        out_specs=[pl.BlockSpec((1, _F, _HW), lambda i: (i, 0, 0)),
                   pl.BlockSpec((1, _F, _HW), lambda i: (i, 0, 0))],
        out_shape=[jax.ShapeDtypeStruct((_BC, _F, _HW), f32),
                   jax.ShapeDtypeStruct((_BC, _F, _HW), f32)],
    )(xf, mask3)

    shape5 = (_B, _C, _F, _H, _W)
    return (out1.reshape(shape5), out2.reshape(shape5))


# speedup vs baseline: 2.5512x; 2.5512x over previous
import jax
import jax.numpy as jnp
from jax.experimental import pallas as pl

_B, _C, _F, _H, _W = 8, 3, 64, 128, 128
_BC = _B * _C
_HW = _H * _W
_KEEP = 32


def _pool_body(x_ref, cw_ref, out_ref, outc_ref):
    i = pl.program_id(0) * _C + pl.program_id(1)
    x = x_ref[0, 0].reshape(_F, _HW)
    s = jnp.sum(x, axis=-1) * (1.0 / _HW)
    prod = jax.lax.dot(cw_ref[...].astype(jnp.bfloat16),
                       x.astype(jnp.bfloat16),
                       preferred_element_type=jnp.float32)
    sc = jnp.sum(prod, axis=-1) * (1.0 / _HW)
    out_ref[pl.ds(i, 1), :] = s.reshape(1, _F)
    outc_ref[pl.ds(i, 1), :] = sc.reshape(1, _F)


def _score_body(pooled_ref, xcv_ref, conv_b_ref, fc_wT_ref,
                fc_b_ref, M1_ref, b1_ref, GG_ref, Mr_ref, br_ref, Ml_ref,
                bl_ref, gamr_ref, betr_ref, A8_ref, B8_ref, P_ref, Q_ref,
                a_ref, mask_ref):
    hi = jax.lax.Precision.HIGHEST
    bf = jnp.bfloat16
    f32 = jnp.float32

    def dot(a, b):
        return jax.lax.dot(a, b, precision=hi)

    def dotb(a, b):
        return jax.lax.dot(a.astype(bf), b.astype(bf),
                           preferred_element_type=f32)

    p = pooled_ref[...]
    x_conv = xcv_ref[...] + conv_b_ref[...]
    scores = jax.nn.sigmoid(dotb(x_conv, fc_wT_ref[...]) + fc_b_ref[...])

    r8 = dot(p, A8_ref[...])
    c8 = dot(p, B8_ref[...])
    xr0 = dotb(M1_ref[...], r8) + b1_ref[...]
    xc0 = dotb(M1_ref[...], c8) + b1_ref[...]

    rs = jnp.sum(xr0, axis=1, keepdims=True) + jnp.sum(xc0, axis=1, keepdims=True)
    mur = dot(GG_ref[...], rs) * (1.0 / 128.0)
    dr = xr0 - mur
    dc = xc0 - mur
    rs2 = (jnp.sum(dr * dr, axis=1, keepdims=True)
           + jnp.sum(dc * dc, axis=1, keepdims=True))
    varr = dot(GG_ref[...], rs2) * (1.0 / 128.0)
    inv = gamr_ref[...] / jnp.sqrt(varr + 1e-5)
    sr = jax.nn.sigmoid(dr * inv + betr_ref[...])
    sc = jax.nn.sigmoid(dc * inv + betr_ref[...])

    ar = jax.nn.sigmoid(dotb(Mr_ref[...], sr) + br_ref[...])
    al = jax.nn.sigmoid(dotb(Ml_ref[...], sc) + bl_ref[...])
    att = dot(ar, P_ref[...]) * dot(al, Q_ref[...])

    a = a_ref[0, 0]
    fin = a * att + (1.0 - a) * scores

    ff = fin[:, :, None]
    fg = fin[:, None, :]
    io_f = jax.lax.broadcasted_iota(jnp.int32, (_BC, _F, _F), 1)
    io_g = jax.lax.broadcasted_iota(jnp.int32, (_BC, _F, _F), 2)
    beats = (fg > ff) | ((fg == ff) & (io_g < io_f))
    cnt = jnp.sum(beats.astype(jnp.float32), axis=2)
    mask_ref[...] = (cnt < float(_KEEP)).astype(jnp.float32)


def _mul_body(x_ref, m_ref, o1_ref, o2_ref):
    o1_ref[...] = x_ref[...] * m_ref[...]
    o2_ref[...] = jnp.zeros_like(o2_ref)


def kernel(x_freq, conv_w, conv_b, conv1_w, conv1_b, convr_w, convr_b,
           convl_w, convl_b, bn_gamma, bn_beta, fc_w, fc_b, a_param):
    f32 = jnp.float32

    pooled, xconv = pl.pallas_call(
        _pool_body,
        grid=(_B, _C),
        in_specs=[pl.BlockSpec((1, 1, _F, _H, _W), lambda b, c: (b, c, 0, 0, 0)),
                  pl.BlockSpec((_F, _F), lambda b, c: (0, 0))],
        out_specs=[pl.BlockSpec((_BC, _F), lambda b, c: (0, 0)),
                   pl.BlockSpec((_BC, _F), lambda b, c: (0, 0))],
        out_shape=[jax.ShapeDtypeStruct((_BC, _F), f32),
                   jax.ShapeDtypeStruct((_BC, _F), f32)],
    )(x_freq, conv_w)

    eyeB = jnp.eye(_B, dtype=f32)
    M1 = jnp.kron(eyeB, conv1_w)
    Mr = jnp.kron(eyeB, convr_w)
    Ml = jnp.kron(eyeB, convl_w)
    b1 = jnp.tile(conv1_b, _B).reshape(_BC, 1)
    br = jnp.tile(convr_b, _B).reshape(_BC, 1)
    bl = jnp.tile(convl_b, _B).reshape(_BC, 1)
    gamr = jnp.tile(bn_gamma, _B).reshape(_BC, 1)
    betr = jnp.tile(bn_beta, _B).reshape(_BC, 1)
    ch = jnp.arange(_BC) % _C
    GG = (ch[:, None] == ch[None, :]).astype(f32)
    q8 = jnp.arange(_F, dtype=jnp.int32)
    A8 = ((q8[:, None] // 8) == jnp.arange(8)[None, :]).astype(f32) / 8.0
    B8 = ((q8[:, None] % 8) == jnp.arange(8)[None, :]).astype(f32) / 8.0
    P = (jnp.arange(8)[:, None] == (q8[None, :] // 8)).astype(f32)
    Q = (jnp.arange(8)[:, None] == (q8[None, :] % 8)).astype(f32)

    mask = pl.pallas_call(
        _score_body,
        out_shape=jax.ShapeDtypeStruct((_BC, _F), f32),
    )(pooled, xconv, conv_b.reshape(1, _F), fc_w.T, fc_b.reshape(1, _F),
      M1, b1, GG, Mr, br, Ml, bl, gamr, betr, A8, B8, P, Q,
      jnp.asarray(a_param, f32).reshape(1, 1))

    mask5 = mask.reshape(_B, _C, _F, 1, 1)
    shape5 = (_B, _C, _F, _H, _W)
    blk5 = (1, 1, _F, _H, _W)
    out1, out2 = pl.pallas_call(
        _mul_body,
        grid=(_B, _C),
        in_specs=[pl.BlockSpec(blk5, lambda b, c: (b, c, 0, 0, 0)),
                  pl.BlockSpec((1, 1, _F, 1, 1), lambda b, c: (b, c, 0, 0, 0))],
        out_specs=[pl.BlockSpec(blk5, lambda b, c: (b, c, 0, 0, 0)),
                   pl.BlockSpec(blk5, lambda b, c: (b, c, 0, 0, 0))],
        out_shape=[jax.ShapeDtypeStruct(shape5, f32),
                   jax.ShapeDtypeStruct(shape5, f32)],
    )(x_freq, mask5)

    return (out1, out2)
